# baseline (speedup 1.0000x reference)
import functools

import numpy as np
import jax
import jax.numpy as jnp
from jax import lax
from jax.experimental import pallas as pl
from jax.experimental.pallas import tpu as pltpu
from jax.experimental.pallas import tpu_sc as plsc

N = 10000
E = 320000
D_CH = 16
D_NODE = 128

E_PAD = 327680
A_ROWS = E_PAD // 8

_z = np.linspace(-10.0, 10.0, 20001)
_dz = _z[1] - _z[0]
_pdf = np.exp(-0.5 * _z ** 2) / np.sqrt(2.0 * np.pi)
_silu_v = _z / (1.0 + np.exp(-_z))
_SILU_NORM = float(1.0 / np.sqrt(np.sum(_silu_v ** 2 * _pdf) * _dz))
_OUT_NORM = float(1.0 / np.sqrt(D_CH * D_NODE))

TE = 2560


def _mlp_body(ef_ref, ea_ref, w1_ref, w2_ref, w3_ref, w4_ref, out_ref):
    def _act(x):
        return x * (1.0 / (1.0 + jnp.exp(-x))) * _SILU_NORM

    cdim = (((0,), (0,)), ((), ()))
    x = ef_ref[...]
    h = _act(lax.dot_general(w1_ref[...], x, cdim,
                             preferred_element_type=jnp.float32))
    h = _act(lax.dot_general(w2_ref[...], h, cdim,
                             preferred_element_type=jnp.float32))
    h = _act(lax.dot_general(w3_ref[...], h, cdim,
                             preferred_element_type=jnp.float32))
    h = h * ea_ref[...]
    out_e = lax.dot_general(h, w4_ref[...], cdim,
                            preferred_element_type=jnp.float32)
    out_e = jnp.where(pl.program_id(0) < E // TE, out_e, 0.0)
    for sb in range(TE // 512):
        for j in range(8):
            base = 512 * sb + 64 * j
            out_ref[sb, :, 16 * j:16 * (j + 1)] = out_e[base:base + 64, :]


def _edge_mlp(edge_feats_t, edge_attrs_t, w1, w2, w3, w4):
    grid = (E_PAD // TE,)
    return pl.pallas_call(
        _mlp_body,
        grid=grid,
        in_specs=[
            pl.BlockSpec((16, TE), lambda i: (0, jnp.minimum(i, E // TE - 1))),
            pl.BlockSpec((1, TE), lambda i: (0, jnp.minimum(i, E // TE - 1))),
            pl.BlockSpec((16, 64), lambda i: (0, 0)),
            pl.BlockSpec((64, 64), lambda i: (0, 0)),
            pl.BlockSpec((64, 64), lambda i: (0, 0)),
            pl.BlockSpec((64, 16), lambda i: (0, 0)),
        ],
        out_specs=pl.BlockSpec((TE // 512, 64, 128), lambda i: (i, 0, 0)),
        out_shape=jax.ShapeDtypeStruct((E_PAD // 512, 64, 128), jnp.float32),
    )(edge_feats_t, edge_attrs_t, w1, w2, w3, w4)


NC = 2
NS = 16
NW = NC * NS
EPW = E_PAD // NW
CH = 128
SCH = 512
NCHK = EPW // CH
NSUP = EPW // SCH
N_PAD = 10240
NSTRIPE = N_PAD // NS


def _sc_scatter(a_packed, snd, rcv, charges_pad):
    mesh = plsc.VectorSubcoreMesh(core_axis_name="c", subcore_axis_name="s")

    @functools.partial(
        pl.kernel,
        mesh=mesh,
        out_type=jax.ShapeDtypeStruct((NC, N_PAD, 16), jnp.float32),
        scratch_types=[
            pltpu.VMEM((NCHK, CH), jnp.int32),
            pltpu.VMEM((NCHK, CH), jnp.int32),
            pltpu.VMEM((64, 128), jnp.float32),
            pltpu.VMEM((64, 128), jnp.float32),
            pltpu.VMEM((SCH, 16), jnp.float32),
            pltpu.VMEM((SCH, 16), jnp.float32),
            pltpu.VMEM((SCH, 16), jnp.float32),
            pltpu.VMEM((SCH, 16), jnp.float32),
            pltpu.VMEM((NSTRIPE, 16), jnp.float32),
            pltpu.VMEM_SHARED((N_PAD, 16), jnp.float32),
            pltpu.VMEM_SHARED((N_PAD, 16), jnp.float32),
            pltpu.SemaphoreType.DMA,
            pltpu.SemaphoreType.DMA,
            pltpu.SemaphoreType.DMA,
            pltpu.SemaphoreType.DMA,
            pltpu.SemaphoreType.DMA,
            pltpu.SemaphoreType.DMA,
        ],
        compiler_params=pltpu.CompilerParams(use_tc_tiling_on_sc=False,
                                             needs_layout_passes=False),
    )
    def scat(a_hbm, snd_hbm, rcv_hbm, c_hbm, out_hbm,
             snd_v, rcv_v, a0, a1, g0, g1, p0, p1, z_v, c_sh, s_sh,
             sa0, sa1, sg0, sg1, ss0, ss1):
        cid = lax.axis_index("c")
        sid = lax.axis_index("s")
        wid = cid * NS + sid
        stripe = pl.ds(sid * NSTRIPE, NSTRIPE)

        pltpu.sync_copy(c_hbm.at[stripe], z_v)
        pltpu.sync_copy(z_v, c_sh.at[stripe])

        def zbody(r, carry):
            z_v[r] = jnp.zeros((16,), jnp.float32)
            return carry

        lax.fori_loop(0, NSTRIPE, zbody, 0)
        pltpu.sync_copy(z_v, s_sh.at[stripe])

        pltpu.sync_copy(snd_hbm.at[wid], snd_v)
        pltpu.sync_copy(rcv_hbm.at[wid], rcv_v)
        plsc.subcore_barrier()

        asb = wid * NSUP
        bufs = ((a0, g0, p0, sa0, sg0, ss0), (a1, g1, p1, sa1, sg1, ss1))

        def start(s, a_b, g_b, sa_b, sg_b):
            pltpu.async_copy(a_hbm.at[asb + s], a_b, sa_b)
            for q in range(4):
                pltpu.async_copy(c_sh.at[snd_v.at[4 * s + q]],
                                 g_b.at[pl.ds(128 * q, 128)], sg_b)

        start(0, a0, g0, sa0, sg0)
        start(1, a1, g1, sa1, sg1)

        def outer(i, carry):
            for b, (a_b, g_b, p_b, sa_b, sg_b, ss_b) in enumerate(bufs):
                s = 2 * i + b
                pltpu.make_async_copy(a_hbm.at[asb + s], a_b, sa_b).wait()
                for q in range(4):
                    pltpu.make_async_copy(c_sh.at[snd_v.at[4 * s + q]],
                                          g_b.at[pl.ds(128 * q, 128)],
                                          sg_b).wait()

                def mul(r, c2, a_b=a_b, g_b=g_b, p_b=p_b):
                    for j in range(8):
                        p_b[64 * j + r] = (
                            a_b[r, 16 * j:16 * (j + 1)] * g_b[64 * j + r])
                    return c2

                lax.fori_loop(0, 64, mul, 0, unroll=2)

                for q in range(4):
                    pltpu.sync_copy(p_b.at[pl.ds(128 * q, 128)],
                                    s_sh.at[rcv_v.at[4 * s + q]], add=True)

                @pl.when(s + 2 < NSUP)
                def _prefetch(s=s, a_b=a_b, g_b=g_b, sa_b=sa_b, sg_b=sg_b):
                    start(s + 2, a_b, g_b, sa_b, sg_b)

            return carry

        lax.fori_loop(0, NSUP // 2, outer, 0)
        plsc.subcore_barrier()

        pltpu.sync_copy(s_sh.at[stripe], z_v)
        pltpu.sync_copy(z_v, out_hbm.at[cid, stripe])

    return scat(a_packed, snd, rcv, charges_pad)


TN = 2000


def _readout_body(s_ref, nf_ref, wn_ref, out_ref):
    s = s_ref[0] + s_ref[1]
    t = lax.dot_general(nf_ref[...], wn_ref[...],
                        (((1,), (1,)), ((), ())),
                        preferred_element_type=jnp.float32)
    out_ref[...] = jnp.sum(s * t, axis=1, keepdims=True) * _OUT_NORM


def _readout(s_partial, node_feats, w_node):
    grid = (N // TN,)
    return pl.pallas_call(
        _readout_body,
        grid=grid,
        in_specs=[
            pl.BlockSpec((NC, TN, 16), lambda i: (0, i, 0)),
            pl.BlockSpec((TN, D_NODE), lambda i: (i, 0)),
            pl.BlockSpec((16, D_NODE), lambda i: (0, 0)),
        ],
        out_specs=pl.BlockSpec((TN, 1), lambda i: (i, 0)),
        out_shape=jax.ShapeDtypeStruct((N, 1), jnp.float32),
    )(s_partial, node_feats, w_node)


def kernel(node_feats, charges_0, charges_induced, edge_feats, edge_attrs,
           field_feats, edge_index, batch, W1, W2, W3, W4, w_node):
    w1 = W1 * (1.0 / np.sqrt(W1.shape[0]))
    w2 = W2 * (1.0 / np.sqrt(W2.shape[0]))
    w3 = W3 * (1.0 / np.sqrt(W3.shape[0]))
    w4 = W4 * (1.0 / np.sqrt(W4.shape[0]))
    a = _edge_mlp(edge_feats.T, edge_attrs.T, w1, w2, w3, w4)
    idx = edge_index.astype(jnp.int32)
    idx = jnp.pad(idx, ((0, 0), (0, E_PAD - E)))
    snd = idx[0].reshape(NW, NCHK, CH)
    rcv = idx[1].reshape(NW, NCHK, CH)
    c_pad = jnp.pad(charges_induced, ((0, N_PAD - N), (0, 0)))
    s_partial = _sc_scatter(a, snd, rcv, c_pad)
    energy = _readout(s_partial, node_feats, w_node)
    return energy.reshape(N)


# speedup vs baseline: 1.0864x; 1.0864x over previous
import functools

import numpy as np
import jax
import jax.numpy as jnp
from jax import lax
from jax.experimental import pallas as pl
from jax.experimental.pallas import tpu as pltpu
from jax.experimental.pallas import tpu_sc as plsc

N = 10000
E = 320000
D_CH = 16
D_NODE = 128

E_PAD = 327680
NSL = 2
E_SL = E_PAD // NSL

_z = np.linspace(-10.0, 10.0, 20001)
_dz = _z[1] - _z[0]
_pdf = np.exp(-0.5 * _z ** 2) / np.sqrt(2.0 * np.pi)
_silu_v = _z / (1.0 + np.exp(-_z))
_SILU_NORM = float(1.0 / np.sqrt(np.sum(_silu_v ** 2 * _pdf) * _dz))
_OUT_NORM = float(1.0 / np.sqrt(D_CH * D_NODE))

TE = 2560


def _mlp_body(ef_ref, ea_ref, w1_ref, w2_ref, w3_ref, w4_ref, out_ref, *, gbase):
    def _act(x):
        return x * (1.0 / (1.0 + jnp.exp(-x))) * _SILU_NORM

    cdim = (((0,), (0,)), ((), ()))
    x = ef_ref[...]
    h = _act(lax.dot_general(w1_ref[...], x, cdim,
                             preferred_element_type=jnp.float32))
    h = _act(lax.dot_general(w2_ref[...], h, cdim,
                             preferred_element_type=jnp.float32))
    h = _act(lax.dot_general(w3_ref[...], h, cdim,
                             preferred_element_type=jnp.float32))
    h = h * ea_ref[...]
    out_e = lax.dot_general(h, w4_ref[...], cdim,
                            preferred_element_type=jnp.float32)
    out_e = jnp.where(gbase + pl.program_id(0) < E // TE, out_e, 0.0)
    for sb in range(TE // 512):
        for j in range(8):
            base = 512 * sb + 64 * j
            out_ref[sb, :, 16 * j:16 * (j + 1)] = out_e[base:base + 64, :]


def _edge_mlp(edge_feats_t, edge_attrs_t, w1, w2, w3, w4, kslice):
    grid = (E_SL // TE,)
    gb = kslice * (E_SL // TE)
    return pl.pallas_call(
        functools.partial(_mlp_body, gbase=gb),
        grid=grid,
        in_specs=[
            pl.BlockSpec((16, TE), lambda i: (0, jnp.minimum(gb + i, E // TE - 1))),
            pl.BlockSpec((1, TE), lambda i: (0, jnp.minimum(gb + i, E // TE - 1))),
            pl.BlockSpec((16, 64), lambda i: (0, 0)),
            pl.BlockSpec((64, 64), lambda i: (0, 0)),
            pl.BlockSpec((64, 64), lambda i: (0, 0)),
            pl.BlockSpec((64, 16), lambda i: (0, 0)),
        ],
        out_specs=pl.BlockSpec((TE // 512, 64, 128), lambda i: (i, 0, 0)),
        out_shape=jax.ShapeDtypeStruct((E_SL // 512, 64, 128), jnp.float32),
    )(edge_feats_t, edge_attrs_t, w1, w2, w3, w4)


NC = 2
NS = 16
NW = NC * NS
EPW = E_SL // NW
CH = 128
SCH = 512
NCHK = EPW // CH
NSUP = EPW // SCH
N_PAD = 10240
NSTRIPE = N_PAD // NS


def _sc_scatter(a_packed, snd, rcv, charges_pad):
    mesh = plsc.VectorSubcoreMesh(core_axis_name="c", subcore_axis_name="s")

    @functools.partial(
        pl.kernel,
        mesh=mesh,
        out_type=jax.ShapeDtypeStruct((NC, N_PAD, 16), jnp.float32),
        scratch_types=[
            pltpu.VMEM((NCHK, CH), jnp.int32),
            pltpu.VMEM((NCHK, CH), jnp.int32),
            pltpu.VMEM((64, 128), jnp.float32),
            pltpu.VMEM((64, 128), jnp.float32),
            pltpu.VMEM((SCH, 16), jnp.float32),
            pltpu.VMEM((SCH, 16), jnp.float32),
            pltpu.VMEM((SCH, 16), jnp.float32),
            pltpu.VMEM((SCH, 16), jnp.float32),
            pltpu.VMEM((NSTRIPE, 16), jnp.float32),
            pltpu.VMEM_SHARED((N_PAD, 16), jnp.float32),
            pltpu.VMEM_SHARED((N_PAD, 16), jnp.float32),
            pltpu.SemaphoreType.DMA,
            pltpu.SemaphoreType.DMA,
            pltpu.SemaphoreType.DMA,
            pltpu.SemaphoreType.DMA,
            pltpu.SemaphoreType.DMA,
            pltpu.SemaphoreType.DMA,
        ],
        compiler_params=pltpu.CompilerParams(use_tc_tiling_on_sc=False,
                                             needs_layout_passes=False),
    )
    def scat(a_hbm, snd_hbm, rcv_hbm, c_hbm, out_hbm,
             snd_v, rcv_v, a0, a1, g0, g1, p0, p1, z_v, c_sh, s_sh,
             sa0, sa1, sg0, sg1, ss0, ss1):
        cid = lax.axis_index("c")
        sid = lax.axis_index("s")
        wid = cid * NS + sid
        stripe = pl.ds(sid * NSTRIPE, NSTRIPE)

        pltpu.sync_copy(c_hbm.at[stripe], z_v)
        pltpu.sync_copy(z_v, c_sh.at[stripe])

        def zbody(r, carry):
            z_v[r] = jnp.zeros((16,), jnp.float32)
            return carry

        lax.fori_loop(0, NSTRIPE, zbody, 0)
        pltpu.sync_copy(z_v, s_sh.at[stripe])

        pltpu.sync_copy(snd_hbm.at[wid], snd_v)
        pltpu.sync_copy(rcv_hbm.at[wid], rcv_v)
        plsc.subcore_barrier()

        asb = wid * NSUP
        bufs = ((a0, g0, p0, sa0, sg0, ss0), (a1, g1, p1, sa1, sg1, ss1))

        def start(s, a_b, g_b, sa_b, sg_b):
            pltpu.async_copy(a_hbm.at[asb + s], a_b, sa_b)
            for q in range(4):
                pltpu.async_copy(c_sh.at[snd_v.at[4 * s + q]],
                                 g_b.at[pl.ds(128 * q, 128)], sg_b)

        start(0, a0, g0, sa0, sg0)
        start(1, a1, g1, sa1, sg1)

        def outer(i, carry):
            for b, (a_b, g_b, p_b, sa_b, sg_b, ss_b) in enumerate(bufs):
                s = 2 * i + b
                pltpu.make_async_copy(a_hbm.at[asb + s], a_b, sa_b).wait()
                for q in range(4):
                    pltpu.make_async_copy(c_sh.at[snd_v.at[4 * s + q]],
                                          g_b.at[pl.ds(128 * q, 128)],
                                          sg_b).wait()

                def mul(r, c2, a_b=a_b, g_b=g_b, p_b=p_b):
                    for j in range(8):
                        p_b[64 * j + r] = (
                            a_b[r, 16 * j:16 * (j + 1)] * g_b[64 * j + r])
                    return c2

                lax.fori_loop(0, 64, mul, 0, unroll=2)

                for q in range(4):
                    pltpu.sync_copy(p_b.at[pl.ds(128 * q, 128)],
                                    s_sh.at[rcv_v.at[4 * s + q]], add=True)

                @pl.when(s + 2 < NSUP)
                def _prefetch(s=s, a_b=a_b, g_b=g_b, sa_b=sa_b, sg_b=sg_b):
                    start(s + 2, a_b, g_b, sa_b, sg_b)

            return carry

        lax.fori_loop(0, NSUP // 2, outer, 0)
        plsc.subcore_barrier()

        pltpu.sync_copy(s_sh.at[stripe], z_v)
        pltpu.sync_copy(z_v, out_hbm.at[cid, stripe])

    return scat(a_packed, snd, rcv, charges_pad)


TN = 2000


def _readout_body(s0_ref, s1_ref, nf_ref, wn_ref, out_ref):
    s = (s0_ref[0] + s0_ref[1]) + (s1_ref[0] + s1_ref[1])
    t = lax.dot_general(nf_ref[...], wn_ref[...],
                        (((1,), (1,)), ((), ())),
                        preferred_element_type=jnp.float32)
    out_ref[...] = jnp.sum(s * t, axis=1, keepdims=True) * _OUT_NORM


def _readout(s_par0, s_par1, node_feats, w_node):
    grid = (N // TN,)
    return pl.pallas_call(
        _readout_body,
        grid=grid,
        in_specs=[
            pl.BlockSpec((NC, TN, 16), lambda i: (0, i, 0)),
            pl.BlockSpec((NC, TN, 16), lambda i: (0, i, 0)),
            pl.BlockSpec((TN, D_NODE), lambda i: (i, 0)),
            pl.BlockSpec((16, D_NODE), lambda i: (0, 0)),
        ],
        out_specs=pl.BlockSpec((TN, 1), lambda i: (i, 0)),
        out_shape=jax.ShapeDtypeStruct((N, 1), jnp.float32),
    )(s_par0, s_par1, node_feats, w_node)


def kernel(node_feats, charges_0, charges_induced, edge_feats, edge_attrs,
           field_feats, edge_index, batch, W1, W2, W3, W4, w_node):
    w1 = W1 * (1.0 / np.sqrt(W1.shape[0]))
    w2 = W2 * (1.0 / np.sqrt(W2.shape[0]))
    w3 = W3 * (1.0 / np.sqrt(W3.shape[0]))
    w4 = W4 * (1.0 / np.sqrt(W4.shape[0]))
    idx = edge_index.astype(jnp.int32)
    idx = jnp.pad(idx, ((0, 0), (0, E_PAD - E)))
    snd = idx[0].reshape(NSL, NW, NCHK, CH)
    rcv = idx[1].reshape(NSL, NW, NCHK, CH)
    c_pad = jnp.pad(charges_induced, ((0, N_PAD - N), (0, 0)))
    eft = edge_feats.T
    eat = edge_attrs.T
    s_parts = []
    for k in range(NSL):
        a_k = _edge_mlp(eft, eat, w1, w2, w3, w4, k)
        s_parts.append(_sc_scatter(a_k, snd[k], rcv[k], c_pad))
    energy = _readout(s_parts[0], s_parts[1], node_feats, w_node)
    return energy.reshape(N)
